# baseline (speedup 1.0000x reference)
import functools

import jax
import jax.numpy as jnp
from jax import lax
from jax.experimental import pallas as pl
from jax.experimental.pallas import tpu as pltpu
from jax.experimental.pallas import tpu_sc as plsc

_N = 10000
_E = 160000
_D = 256
_H = 8
_HD = 32
_FF = 1024
_SCALE = float(_HD) ** -0.5

_NC = 2
_NS = 16
_G = 16
_EC = _E // _NS
_NG = _EC // _G
_W = 136
_RPT = _N // _NS
_NB = 2

_R = 400
_NR = _N // _R


def _edge_body(qa, qb, kva, kvb, dst4, src4, zeros, out,
               acc_sh, idx_dst, idx_src, qg, kvg, og,
               semq, semkv, sems):
    c = lax.axis_index("c")
    s = lax.axis_index("s")
    iota16 = lax.iota(jnp.int32, 16)
    zero16 = jnp.zeros((16,), jnp.float32)

    pltpu.sync_copy(dst4.at[s], idx_dst)
    pltpu.sync_copy(src4.at[s], idx_src)

    for b in range(_NB):
        def zrow(r, carry):
            for j in range(_W // 16):
                og[b, r, pl.ds(j * 16, 16)] = zero16
            og[b, r, pl.ds(_W - 16, 16)] = zero16
            return carry
        lax.fori_loop(0, _G, zrow, 0)

    base = s * _RPT
    pltpu.sync_copy(zeros, acc_sh.at[pl.ds(base, _RPT)])
    plsc.subcore_barrier()

    sem_by_slot = tuple(
        (semq[b], semkv[b], sems[b]) for b in range(_NB))

    def issue(g, b):
        sq, skv, _ = sem_by_slot[b]

        @pl.when(c == 0)
        def _():
            pltpu.async_copy(qa.at[idx_dst.at[g]], qg.at[b], sq)
            pltpu.async_copy(kva.at[idx_src.at[g]], kvg.at[b], skv)

        @pl.when(c == 1)
        def _():
            pltpu.async_copy(qb.at[idx_dst.at[g]], qg.at[b], sq)
            pltpu.async_copy(kvb.at[idx_src.at[g]], kvg.at[b], skv)

    def consume(g, b, last):
        sq, skv, ss = sem_by_slot[b]

        @pl.when(g >= _NB)
        def _():
            pltpu.make_async_copy(og.at[b], acc_sh.at[idx_dst.at[g]],
                                  ss).wait()

        pltpu.make_async_copy(qa.at[idx_dst.at[g]], qg.at[b], sq).wait()
        pltpu.make_async_copy(kva.at[idx_src.at[g]], kvg.at[b], skv).wait()

        mask0 = iota16 == 0

        def edge(i, c1):
            es = (4 * i, 4 * i + 1, 4 * i + 2, 4 * i + 3)
            ps = []
            for e in es:
                for h in range(4):
                    p = (qg[b, e, pl.ds(h * 32, 16)] *
                         kvg[b, e, pl.ds(h * 32, 16)]
                         + qg[b, e, pl.ds(h * 32 + 16, 16)] *
                         kvg[b, e, pl.ds(h * 32 + 16, 16)])
                    ps.append(p)
            dots = [jnp.sum(p) for p in ps]
            for j, e in enumerate(es):
                for h in range(4):
                    we = jnp.exp(
                        lax.broadcast_in_dim(dots[4 * j + h], (16,), ()))
                    og[b, e, pl.ds(h * 32, 16)] = (
                        kvg[b, e, pl.ds(128 + h * 32, 16)] * we)
                    og[b, e, pl.ds(h * 32 + 16, 16)] = (
                        kvg[b, e, pl.ds(128 + h * 32 + 16, 16)] * we)
                    plsc.store_scatter(
                        og.at[b],
                        [jnp.full((16,), e, jnp.int32),
                         jnp.full((16,), 128 + h, jnp.int32)],
                        we, mask=mask0)
            return c1

        lax.fori_loop(0, _G // 4, edge, 0)

        pltpu.async_copy(og.at[b], acc_sh.at[idx_dst.at[g]], ss, add=True)

        if not last:
            @pl.when(g + _NB < _NG)
            def _():
                issue(g + _NB, b)

    for b in range(_NB):
        issue(b, b)

    def turn(j, carry):
        for b in range(_NB):
            consume(_NB * j + b, b, False)
        return carry

    lax.fori_loop(0, _NG // _NB, turn, 0)
    consume(_NG - 1, (_NG - 1) % _NB, True)

    for b in range(_NB):
        pltpu.make_async_copy(og.at[b], acc_sh.at[idx_dst.at[0]],
                              sems[b]).wait()
    plsc.subcore_barrier()

    pltpu.sync_copy(acc_sh.at[pl.ds(base, _RPT)],
                    out.at[pl.ds(c * _N + base, _RPT)])


@functools.cache
def _edge_call():
  return pl.kernel(
    _edge_body,
    out_type=jax.ShapeDtypeStruct((2 * _N, _W), jnp.float32),
    mesh=plsc.VectorSubcoreMesh(core_axis_name="c", subcore_axis_name="s",
                                num_cores=_NC, num_subcores=_NS),
    scratch_types=[
        pltpu.VMEM_SHARED((_N, _W), jnp.float32),
        pltpu.VMEM((_NG, _G), jnp.int32),
        pltpu.VMEM((_NG, _G), jnp.int32),
        pltpu.VMEM((_NB, _G, 128), jnp.float32),
        pltpu.VMEM((_NB, _G, 256), jnp.float32),
        pltpu.VMEM((_NB, _G, _W), jnp.float32),
        [pltpu.SemaphoreType.DMA] * _NB,
        [pltpu.SemaphoreType.DMA] * _NB,
        [pltpu.SemaphoreType.DMA] * _NB,
    ],
    compiler_params=pltpu.CompilerParams(use_tc_tiling_on_sc=False,
                                         needs_layout_passes=False),
  )


def _bn_from_stats(x, st, g, be):
    mu = st[0:1, :] * (1.0 / _N)
    var = st[1:2, :] * (1.0 / _N) - mu * mu
    inv = lax.rsqrt(var + 1e-5)
    return (x - mu) * inv * g + be


def _store_qkv(q, k, v, qao, qbo, kvao, kvbo):
    qao[...] = q[:, :128]
    qbo[...] = q[:, 128:]
    kvao[...] = jnp.concatenate([k[:, :128], v[:, :128]], axis=1)
    kvbo[...] = jnp.concatenate([k[:, 128:], v[:, 128:]], axis=1)


def _qkv0_body(x, qW, qb, kW, vW, qao, qbo, kvao, kvbo):
    xb = x[...]
    q = (jnp.dot(xb, qW[...], preferred_element_type=jnp.float32)
         + qb[...]) * _SCALE
    k = jnp.dot(xb, kW[...], preferred_element_type=jnp.float32)
    v = jnp.dot(xb, vW[...], preferred_element_type=jnp.float32)
    _store_qkv(q, k, v, qao, qbo, kvao, kvbo)


def _qkv1_body(z, st, g, be, qW, qb, kW, vW, xo, qao, qbo, kvao, kvbo):
    xb = _bn_from_stats(z[...], st[...], g[...], be[...])
    xo[...] = xb
    q = (jnp.dot(xb, qW[...], preferred_element_type=jnp.float32)
         + qb[...]) * _SCALE
    k = jnp.dot(xb, kW[...], preferred_element_type=jnp.float32)
    v = jnp.dot(xb, vW[...], preferred_element_type=jnp.float32)
    _store_qkv(q, k, v, qao, qbo, kvao, kvbo)


def _post_body(acca, accb, x, oW, ob, yo, sto):
    i = pl.program_id(0)
    pieces = []
    for a in (acca[...], accb[...]):
        for h in range(4):
            num = a[:, 32 * h:32 * h + 32]
            den = a[:, 128 + h][:, None] + 1e-16
            pieces.append(num / den)
    attn = jnp.concatenate(pieces, axis=1)
    y = jnp.dot(attn, oW[...], preferred_element_type=jnp.float32) + ob[...] + x[...]
    yo[...] = y

    @pl.when(i == 0)
    def _():
        sto[...] = jnp.zeros_like(sto)

    sto[...] += jnp.stack([jnp.sum(y, axis=0), jnp.sum(y * y, axis=0)])


def _ffn_body(y, st, w1, b1, w2, b2, g1, be1, zo, sto):
    i = pl.program_id(0)
    xb = _bn_from_stats(y[...], st[...], g1[...], be1[...])
    hh = jnp.maximum(jnp.dot(xb, w1[...], preferred_element_type=jnp.float32)
                     + b1[...], 0.0)
    z = jnp.dot(hh, w2[...], preferred_element_type=jnp.float32) + b2[...] + xb
    zo[...] = z

    @pl.when(i == 0)
    def _():
        sto[...] = jnp.zeros_like(sto)

    sto[...] += jnp.stack([jnp.sum(z, axis=0), jnp.sum(z * z, axis=0)])


def _final_body(z, st, g2, be2, lng, lnb, oo):
    xb = _bn_from_stats(z[...], st[...], g2[...], be2[...])
    mu = jnp.mean(xb, axis=1, keepdims=True)
    d = xb - mu
    var = jnp.mean(d * d, axis=1, keepdims=True)
    oo[...] = d * lax.rsqrt(var + 1e-5) * lng[...] + lnb[...]


def _full(shape):
    return pl.BlockSpec(shape, lambda i: tuple(0 for _ in shape))


def _rows(width):
    return pl.BlockSpec((_R, width), lambda i: (i, 0))


_f32 = jnp.float32


def _call_qkv0(x, qW, qb, kW, vW):
    return pl.pallas_call(
        _qkv0_body,
        grid=(_NR,),
        in_specs=[_rows(_D), _full((_D, _D)), _full((1, _D)),
                  _full((_D, _D)), _full((_D, _D))],
        out_specs=[_rows(128), _rows(128), _rows(256), _rows(256)],
        out_shape=[jax.ShapeDtypeStruct((_N, 128), _f32)] * 2
        + [jax.ShapeDtypeStruct((_N, 256), _f32)] * 2,
    )(x, qW, qb.reshape(1, _D), kW, vW)


def _call_qkv1(z, st, g, be, qW, qb, kW, vW):
    return pl.pallas_call(
        _qkv1_body,
        grid=(_NR,),
        in_specs=[_rows(_D), _full((2, _D)), _full((1, _D)), _full((1, _D)),
                  _full((_D, _D)), _full((1, _D)), _full((_D, _D)),
                  _full((_D, _D))],
        out_specs=[_rows(_D), _rows(128), _rows(128), _rows(256), _rows(256)],
        out_shape=([jax.ShapeDtypeStruct((_N, _D), _f32)]
                   + [jax.ShapeDtypeStruct((_N, 128), _f32)] * 2
                   + [jax.ShapeDtypeStruct((_N, 256), _f32)] * 2),
    )(z, st, g.reshape(1, _D), be.reshape(1, _D), qW, qb.reshape(1, _D), kW, vW)


def _call_post(acc, x, oW, ob):
    return pl.pallas_call(
        _post_body,
        grid=(_NR,),
        in_specs=[pl.BlockSpec((_R, _W), lambda i: (i, 0)),
                  pl.BlockSpec((_R, _W), lambda i: (i + _NR, 0)),
                  _rows(_D), _full((_D, _D)), _full((1, _D))],
        out_specs=[_rows(_D), _full((2, _D))],
        out_shape=[jax.ShapeDtypeStruct((_N, _D), _f32),
                   jax.ShapeDtypeStruct((2, _D), _f32)],
    )(acc, acc, x, oW, ob.reshape(1, _D))


def _call_ffn(y, st, w1, b1, w2, b2, g1, be1):
    return pl.pallas_call(
        _ffn_body,
        grid=(_NR,),
        in_specs=[_rows(_D), _full((2, _D)), _full((_D, _FF)), _full((1, _FF)),
                  _full((_FF, _D)), _full((1, _D)), _full((1, _D)),
                  _full((1, _D))],
        out_specs=[_rows(_D), _full((2, _D))],
        out_shape=[jax.ShapeDtypeStruct((_N, _D), _f32),
                   jax.ShapeDtypeStruct((2, _D), _f32)],
    )(y, st, w1, b1.reshape(1, _FF), w2, b2.reshape(1, _D),
      g1.reshape(1, _D), be1.reshape(1, _D))


def _call_final(z, st, g2, be2, lng, lnb):
    return pl.pallas_call(
        _final_body,
        grid=(_NR,),
        in_specs=[_rows(_D), _full((2, _D)), _full((1, _D)), _full((1, _D)),
                  _full((1, _D)), _full((1, _D))],
        out_specs=_rows(_D),
        out_shape=jax.ShapeDtypeStruct((_N, _D), _f32),
    )(z, st, g2.reshape(1, _D), be2.reshape(1, _D),
      lng.reshape(1, _D), lnb.reshape(1, _D))


def _edge(qkv, dst3, src3, zeros):
    return _edge_call()(*qkv, dst3, src3, zeros)


def kernel(src, edge_index,
           l0_qW, l0_qb, l0_kW, l0_vW, l0_oW, l0_ob, l0_w1, l0_b1, l0_w2,
           l0_b2, l0_g1, l0_be1, l0_g2, l0_be2,
           l1_qW, l1_qb, l1_kW, l1_vW, l1_oW, l1_ob, l1_w1, l1_b1, l1_w2,
           l1_b2, l1_g1, l1_be1, l1_g2, l1_be2,
           ln_g, ln_b):
    dst3 = edge_index[1].reshape(_NS, _NG, _G)
    src3 = edge_index[0].reshape(_NS, _NG, _G)
    zeros = jnp.zeros((_RPT, _W), jnp.float32)

    qkv = _call_qkv0(src, l0_qW, l0_qb, l0_kW, l0_vW)
    acc = _edge(qkv, dst3, src3, zeros)
    y, st = _call_post(acc, src, l0_oW, l0_ob)
    z, st = _call_ffn(y, st, l0_w1, l0_b1, l0_w2, l0_b2, l0_g1, l0_be1)

    xb, *qkv = _call_qkv1(z, st, l0_g2, l0_be2, l1_qW, l1_qb, l1_kW, l1_vW)
    acc = _edge(qkv, dst3, src3, zeros)
    y, st = _call_post(acc, xb, l1_oW, l1_ob)
    z, st = _call_ffn(y, st, l1_w1, l1_b1, l1_w2, l1_b2, l1_g1, l1_be1)

    return _call_final(z, st, l1_g2, l1_be2, ln_g, ln_b)


# speedup vs baseline: 1.0171x; 1.0171x over previous
import functools

import jax
import jax.numpy as jnp
from jax import lax
from jax.experimental import pallas as pl
from jax.experimental.pallas import tpu as pltpu
from jax.experimental.pallas import tpu_sc as plsc

_N = 10000
_E = 160000
_D = 256
_H = 8
_HD = 32
_FF = 1024
_SCALE = float(_HD) ** -0.5

_NC = 2
_NS = 16
_G = 16
_EC = _E // _NS
_NG = _EC // _G
_W = 136
_RPT = _N // _NS
_NB = 2

_R = 400
_NR = _N // _R


def _edge_body(qa, qb, ka, kb, va, vb, dst4, src4, zeros, out,
               acc_sh, idx_dst, idx_src, qg, kg, vg, og,
               semq, semk, semv, sems):
    c = lax.axis_index("c")
    s = lax.axis_index("s")
    iota16 = lax.iota(jnp.int32, 16)
    zero16 = jnp.zeros((16,), jnp.float32)

    pltpu.sync_copy(dst4.at[s], idx_dst)
    pltpu.sync_copy(src4.at[s], idx_src)

    for b in range(_NB):
        def zrow(r, carry):
            for j in range(_W // 16):
                og[b, r, pl.ds(j * 16, 16)] = zero16
            og[b, r, pl.ds(_W - 16, 16)] = zero16
            return carry
        lax.fori_loop(0, _G, zrow, 0)

    base = s * _RPT
    pltpu.sync_copy(zeros, acc_sh.at[pl.ds(base, _RPT)])
    plsc.subcore_barrier()

    sem_by_slot = tuple(
        (semq[b], semk[b], semv[b], sems[b]) for b in range(_NB))

    def issue(g, b):
        sq, sk, sv_, _ = sem_by_slot[b]

        @pl.when(c == 0)
        def _():
            pltpu.async_copy(qa.at[idx_dst.at[g]], qg.at[b], sq)
            pltpu.async_copy(ka.at[idx_src.at[g]], kg.at[b], sk)
            pltpu.async_copy(va.at[idx_src.at[g]], vg.at[b], sv_)

        @pl.when(c == 1)
        def _():
            pltpu.async_copy(qb.at[idx_dst.at[g]], qg.at[b], sq)
            pltpu.async_copy(kb.at[idx_src.at[g]], kg.at[b], sk)
            pltpu.async_copy(vb.at[idx_src.at[g]], vg.at[b], sv_)

    def consume(g, b, last):
        sq, sk, sv_, ss = sem_by_slot[b]

        @pl.when(g >= _NB)
        def _():
            pltpu.make_async_copy(og.at[b], acc_sh.at[idx_dst.at[g]],
                                  ss).wait()

        pltpu.make_async_copy(qa.at[idx_dst.at[g]], qg.at[b], sq).wait()
        pltpu.make_async_copy(ka.at[idx_src.at[g]], kg.at[b], sk).wait()
        pltpu.make_async_copy(va.at[idx_src.at[g]], vg.at[b], sv_).wait()

        mask0 = iota16 == 0

        def edge(i, c1):
            es = (4 * i, 4 * i + 1, 4 * i + 2, 4 * i + 3)
            ps = []
            for e in es:
                for h in range(4):
                    p = (qg[b, e, pl.ds(h * 32, 16)] *
                         kg[b, e, pl.ds(h * 32, 16)]
                         + qg[b, e, pl.ds(h * 32 + 16, 16)] *
                         kg[b, e, pl.ds(h * 32 + 16, 16)])
                    ps.append(p)
            dots = [jnp.sum(p) for p in ps]
            for j, e in enumerate(es):
                for h in range(4):
                    we = jnp.exp(
                        lax.broadcast_in_dim(dots[4 * j + h], (16,), ()))
                    og[b, e, pl.ds(h * 32, 16)] = (
                        vg[b, e, pl.ds(h * 32, 16)] * we)
                    og[b, e, pl.ds(h * 32 + 16, 16)] = (
                        vg[b, e, pl.ds(h * 32 + 16, 16)] * we)
                    plsc.store_scatter(
                        og.at[b],
                        [jnp.full((16,), e, jnp.int32),
                         jnp.full((16,), 128 + h, jnp.int32)],
                        we, mask=mask0)
            return c1

        lax.fori_loop(0, _G // 4, edge, 0)

        pltpu.async_copy(og.at[b], acc_sh.at[idx_dst.at[g]], ss, add=True)

        if not last:
            @pl.when(g + _NB < _NG)
            def _():
                issue(g + _NB, b)

    for b in range(_NB):
        issue(b, b)

    def turn(j, carry):
        for b in range(_NB):
            consume(_NB * j + b, b, False)
        return carry

    lax.fori_loop(0, _NG // _NB, turn, 0)
    consume(_NG - 1, (_NG - 1) % _NB, True)

    for b in range(_NB):
        pltpu.make_async_copy(og.at[b], acc_sh.at[idx_dst.at[0]],
                              sems[b]).wait()
    plsc.subcore_barrier()

    pltpu.sync_copy(acc_sh.at[pl.ds(base, _RPT)],
                    out.at[pl.ds(c * _N + base, _RPT)])


@functools.cache
def _edge_call():
  return pl.kernel(
    _edge_body,
    out_type=jax.ShapeDtypeStruct((2 * _N, _W), jnp.float32),
    mesh=plsc.VectorSubcoreMesh(core_axis_name="c", subcore_axis_name="s",
                                num_cores=_NC, num_subcores=_NS),
    scratch_types=[
        pltpu.VMEM_SHARED((_N, _W), jnp.float32),
        pltpu.VMEM((_NG, _G), jnp.int32),
        pltpu.VMEM((_NG, _G), jnp.int32),
        pltpu.VMEM((_NB, _G, 128), jnp.float32),
        pltpu.VMEM((_NB, _G, 128), jnp.float32),
        pltpu.VMEM((_NB, _G, 128), jnp.float32),
        pltpu.VMEM((_NB, _G, _W), jnp.float32),
        [pltpu.SemaphoreType.DMA] * _NB,
        [pltpu.SemaphoreType.DMA] * _NB,
        [pltpu.SemaphoreType.DMA] * _NB,
        [pltpu.SemaphoreType.DMA] * _NB,
    ],
    compiler_params=pltpu.CompilerParams(use_tc_tiling_on_sc=False,
                                         needs_layout_passes=False),
  )


def _bn_from_stats(x, st, g, be):
    mu = st[0:1, :] * (1.0 / _N)
    var = st[1:2, :] * (1.0 / _N) - mu * mu
    inv = lax.rsqrt(var + 1e-5)
    return (x - mu) * inv * g + be


def _store_qkv(q, k, v, qao, qbo, kao, kbo, vao, vbo):
    qao[...] = q[:, :128]
    qbo[...] = q[:, 128:]
    kao[...] = k[:, :128]
    kbo[...] = k[:, 128:]
    vao[...] = v[:, :128]
    vbo[...] = v[:, 128:]


def _qkv0_body(x, qW, qb, kW, vW, qao, qbo, kao, kbo, vao, vbo):
    xb = x[...]
    q = (jnp.dot(xb, qW[...], preferred_element_type=jnp.float32)
         + qb[...]) * _SCALE
    k = jnp.dot(xb, kW[...], preferred_element_type=jnp.float32)
    v = jnp.dot(xb, vW[...], preferred_element_type=jnp.float32)
    _store_qkv(q, k, v, qao, qbo, kao, kbo, vao, vbo)


def _qkv1_body(z, st, g, be, qW, qb, kW, vW,
               xo, qao, qbo, kao, kbo, vao, vbo):
    xb = _bn_from_stats(z[...], st[...], g[...], be[...])
    xo[...] = xb
    q = (jnp.dot(xb, qW[...], preferred_element_type=jnp.float32)
         + qb[...]) * _SCALE
    k = jnp.dot(xb, kW[...], preferred_element_type=jnp.float32)
    v = jnp.dot(xb, vW[...], preferred_element_type=jnp.float32)
    _store_qkv(q, k, v, qao, qbo, kao, kbo, vao, vbo)


def _post_body(acca, accb, x, oW, ob, yo, sto):
    i = pl.program_id(0)
    pieces = []
    for a in (acca[...], accb[...]):
        for h in range(4):
            num = a[:, 32 * h:32 * h + 32]
            den = a[:, 128 + h][:, None] + 1e-16
            pieces.append(num / den)
    attn = jnp.concatenate(pieces, axis=1)
    y = jnp.dot(attn, oW[...], preferred_element_type=jnp.float32) + ob[...] + x[...]
    yo[...] = y

    @pl.when(i == 0)
    def _():
        sto[...] = jnp.zeros_like(sto)

    sto[...] += jnp.stack([jnp.sum(y, axis=0), jnp.sum(y * y, axis=0)])


def _ffn_body(y, st, w1, b1, w2, b2, g1, be1, zo, sto):
    i = pl.program_id(0)
    xb = _bn_from_stats(y[...], st[...], g1[...], be1[...])
    hh = jnp.maximum(jnp.dot(xb, w1[...], preferred_element_type=jnp.float32)
                     + b1[...], 0.0)
    z = jnp.dot(hh, w2[...], preferred_element_type=jnp.float32) + b2[...] + xb
    zo[...] = z

    @pl.when(i == 0)
    def _():
        sto[...] = jnp.zeros_like(sto)

    sto[...] += jnp.stack([jnp.sum(z, axis=0), jnp.sum(z * z, axis=0)])


def _final_body(z, st, g2, be2, lng, lnb, oo):
    xb = _bn_from_stats(z[...], st[...], g2[...], be2[...])
    mu = jnp.mean(xb, axis=1, keepdims=True)
    d = xb - mu
    var = jnp.mean(d * d, axis=1, keepdims=True)
    oo[...] = d * lax.rsqrt(var + 1e-5) * lng[...] + lnb[...]


def _full(shape):
    return pl.BlockSpec(shape, lambda i: tuple(0 for _ in shape))


def _rows(width):
    return pl.BlockSpec((_R, width), lambda i: (i, 0))


_f32 = jnp.float32


def _call_qkv0(x, qW, qb, kW, vW):
    return pl.pallas_call(
        _qkv0_body,
        grid=(_NR,),
        in_specs=[_rows(_D), _full((_D, _D)), _full((1, _D)),
                  _full((_D, _D)), _full((_D, _D))],
        out_specs=[_rows(128)] * 6,
        out_shape=[jax.ShapeDtypeStruct((_N, 128), _f32)] * 6,
    )(x, qW, qb.reshape(1, _D), kW, vW)


def _call_qkv1(z, st, g, be, qW, qb, kW, vW):
    return pl.pallas_call(
        _qkv1_body,
        grid=(_NR,),
        in_specs=[_rows(_D), _full((2, _D)), _full((1, _D)), _full((1, _D)),
                  _full((_D, _D)), _full((1, _D)), _full((_D, _D)),
                  _full((_D, _D))],
        out_specs=[_rows(_D)] + [_rows(128)] * 6,
        out_shape=([jax.ShapeDtypeStruct((_N, _D), _f32)]
                   + [jax.ShapeDtypeStruct((_N, 128), _f32)] * 6),
    )(z, st, g.reshape(1, _D), be.reshape(1, _D), qW, qb.reshape(1, _D), kW, vW)


def _call_post(acc, x, oW, ob):
    return pl.pallas_call(
        _post_body,
        grid=(_NR,),
        in_specs=[pl.BlockSpec((_R, _W), lambda i: (i, 0)),
                  pl.BlockSpec((_R, _W), lambda i: (i + _NR, 0)),
                  _rows(_D), _full((_D, _D)), _full((1, _D))],
        out_specs=[_rows(_D), _full((2, _D))],
        out_shape=[jax.ShapeDtypeStruct((_N, _D), _f32),
                   jax.ShapeDtypeStruct((2, _D), _f32)],
    )(acc, acc, x, oW, ob.reshape(1, _D))


def _call_ffn(y, st, w1, b1, w2, b2, g1, be1):
    return pl.pallas_call(
        _ffn_body,
        grid=(_NR,),
        in_specs=[_rows(_D), _full((2, _D)), _full((_D, _FF)), _full((1, _FF)),
                  _full((_FF, _D)), _full((1, _D)), _full((1, _D)),
                  _full((1, _D))],
        out_specs=[_rows(_D), _full((2, _D))],
        out_shape=[jax.ShapeDtypeStruct((_N, _D), _f32),
                   jax.ShapeDtypeStruct((2, _D), _f32)],
    )(y, st, w1, b1.reshape(1, _FF), w2, b2.reshape(1, _D),
      g1.reshape(1, _D), be1.reshape(1, _D))


def _call_final(z, st, g2, be2, lng, lnb):
    return pl.pallas_call(
        _final_body,
        grid=(_NR,),
        in_specs=[_rows(_D), _full((2, _D)), _full((1, _D)), _full((1, _D)),
                  _full((1, _D)), _full((1, _D))],
        out_specs=_rows(_D),
        out_shape=jax.ShapeDtypeStruct((_N, _D), _f32),
    )(z, st, g2.reshape(1, _D), be2.reshape(1, _D),
      lng.reshape(1, _D), lnb.reshape(1, _D))


def _edge(qkv, dst3, src3, zeros):
    return _edge_call()(*qkv, dst3, src3, zeros)


def kernel(src, edge_index,
           l0_qW, l0_qb, l0_kW, l0_vW, l0_oW, l0_ob, l0_w1, l0_b1, l0_w2,
           l0_b2, l0_g1, l0_be1, l0_g2, l0_be2,
           l1_qW, l1_qb, l1_kW, l1_vW, l1_oW, l1_ob, l1_w1, l1_b1, l1_w2,
           l1_b2, l1_g1, l1_be1, l1_g2, l1_be2,
           ln_g, ln_b):
    dst3 = edge_index[1].reshape(_NS, _NG, _G)
    src3 = edge_index[0].reshape(_NS, _NG, _G)
    zeros = jnp.zeros((_RPT, _W), jnp.float32)

    qkv = _call_qkv0(src, l0_qW, l0_qb, l0_kW, l0_vW)
    acc = _edge(qkv, dst3, src3, zeros)
    y, st = _call_post(acc, src, l0_oW, l0_ob)
    z, st = _call_ffn(y, st, l0_w1, l0_b1, l0_w2, l0_b2, l0_g1, l0_be1)

    xb, *qkv = _call_qkv1(z, st, l0_g2, l0_be2, l1_qW, l1_qb, l1_kW, l1_vW)
    acc = _edge(qkv, dst3, src3, zeros)
    y, st = _call_post(acc, xb, l1_oW, l1_ob)
    z, st = _call_ffn(y, st, l1_w1, l1_b1, l1_w2, l1_b2, l1_g1, l1_be1)

    return _call_final(z, st, l1_g2, l1_be2, ln_g, ln_b)


# speedup vs baseline: 1.0613x; 1.0434x over previous
import functools

import jax
import jax.numpy as jnp
from jax import lax
from jax.experimental import pallas as pl
from jax.experimental.pallas import tpu as pltpu
from jax.experimental.pallas import tpu_sc as plsc

_N = 10000
_E = 160000
_D = 256
_H = 8
_HD = 32
_FF = 1024
_SCALE = float(_HD) ** -0.5

_NC = 2
_NS = 16
_G = 16
_EC = _E // _NS
_NG = _EC // _G
_W = 136
_RPT = _N // _NS
_NB = 2

_R = 1000
_NR = _N // _R


def _edge_body(qa, qb, ka, kb, va, vb, dst4, src4, zeros, out,
               acc_sh, idx_dst, idx_src, qg, kg, vg, og,
               semq, semk, semv, sems):
    c = lax.axis_index("c")
    s = lax.axis_index("s")
    iota16 = lax.iota(jnp.int32, 16)
    zero16 = jnp.zeros((16,), jnp.float32)

    pltpu.sync_copy(dst4.at[s], idx_dst)
    pltpu.sync_copy(src4.at[s], idx_src)

    for b in range(_NB):
        def zrow(r, carry):
            for j in range(_W // 16):
                og[b, r, pl.ds(j * 16, 16)] = zero16
            og[b, r, pl.ds(_W - 16, 16)] = zero16
            return carry
        lax.fori_loop(0, _G, zrow, 0)

    base = s * _RPT
    pltpu.sync_copy(zeros, acc_sh.at[pl.ds(base, _RPT)])
    plsc.subcore_barrier()

    sem_by_slot = tuple(
        (semq[b], semk[b], semv[b], sems[b]) for b in range(_NB))

    def issue(g, b):
        sq, sk, sv_, _ = sem_by_slot[b]

        @pl.when(c == 0)
        def _():
            pltpu.async_copy(qa.at[idx_dst.at[g]], qg.at[b], sq)
            pltpu.async_copy(ka.at[idx_src.at[g]], kg.at[b], sk)
            pltpu.async_copy(va.at[idx_src.at[g]], vg.at[b], sv_)

        @pl.when(c == 1)
        def _():
            pltpu.async_copy(qb.at[idx_dst.at[g]], qg.at[b], sq)
            pltpu.async_copy(kb.at[idx_src.at[g]], kg.at[b], sk)
            pltpu.async_copy(vb.at[idx_src.at[g]], vg.at[b], sv_)

    def consume(g, b, last):
        sq, sk, sv_, ss = sem_by_slot[b]

        @pl.when(g >= _NB)
        def _():
            pltpu.make_async_copy(og.at[b], acc_sh.at[idx_dst.at[g]],
                                  ss).wait()

        pltpu.make_async_copy(qa.at[idx_dst.at[g]], qg.at[b], sq).wait()
        pltpu.make_async_copy(ka.at[idx_src.at[g]], kg.at[b], sk).wait()
        pltpu.make_async_copy(va.at[idx_src.at[g]], vg.at[b], sv_).wait()

        mask0 = iota16 == 0

        def edge(i, c1):
            es = (4 * i, 4 * i + 1, 4 * i + 2, 4 * i + 3)
            ps = []
            for e in es:
                for h in range(4):
                    p = (qg[b, e, pl.ds(h * 32, 16)] *
                         kg[b, e, pl.ds(h * 32, 16)]
                         + qg[b, e, pl.ds(h * 32 + 16, 16)] *
                         kg[b, e, pl.ds(h * 32 + 16, 16)])
                    ps.append(p)
            dots = [jnp.sum(p) for p in ps]
            for j, e in enumerate(es):
                for h in range(4):
                    we = jnp.exp(
                        lax.broadcast_in_dim(dots[4 * j + h], (16,), ()))
                    og[b, e, pl.ds(h * 32, 16)] = (
                        vg[b, e, pl.ds(h * 32, 16)] * we)
                    og[b, e, pl.ds(h * 32 + 16, 16)] = (
                        vg[b, e, pl.ds(h * 32 + 16, 16)] * we)
                    plsc.store_scatter(
                        og.at[b],
                        [jnp.full((16,), e, jnp.int32),
                         jnp.full((16,), 128 + h, jnp.int32)],
                        we, mask=mask0)
            return c1

        lax.fori_loop(0, _G // 4, edge, 0)

        pltpu.async_copy(og.at[b], acc_sh.at[idx_dst.at[g]], ss, add=True)

        if not last:
            @pl.when(g + _NB < _NG)
            def _():
                issue(g + _NB, b)

    for b in range(_NB):
        issue(b, b)

    def turn(j, carry):
        for b in range(_NB):
            consume(_NB * j + b, b, False)
        return carry

    lax.fori_loop(0, _NG // _NB, turn, 0)
    consume(_NG - 1, (_NG - 1) % _NB, True)

    for b in range(_NB):
        pltpu.make_async_copy(og.at[b], acc_sh.at[idx_dst.at[0]],
                              sems[b]).wait()
    plsc.subcore_barrier()

    pltpu.sync_copy(acc_sh.at[pl.ds(base, _RPT)],
                    out.at[pl.ds(c * _N + base, _RPT)])


@functools.cache
def _edge_call():
  return pl.kernel(
    _edge_body,
    out_type=jax.ShapeDtypeStruct((2 * _N, _W), jnp.float32),
    mesh=plsc.VectorSubcoreMesh(core_axis_name="c", subcore_axis_name="s",
                                num_cores=_NC, num_subcores=_NS),
    scratch_types=[
        pltpu.VMEM_SHARED((_N, _W), jnp.float32),
        pltpu.VMEM((_NG, _G), jnp.int32),
        pltpu.VMEM((_NG, _G), jnp.int32),
        pltpu.VMEM((_NB, _G, 128), jnp.float32),
        pltpu.VMEM((_NB, _G, 128), jnp.float32),
        pltpu.VMEM((_NB, _G, 128), jnp.float32),
        pltpu.VMEM((_NB, _G, _W), jnp.float32),
        [pltpu.SemaphoreType.DMA] * _NB,
        [pltpu.SemaphoreType.DMA] * _NB,
        [pltpu.SemaphoreType.DMA] * _NB,
        [pltpu.SemaphoreType.DMA] * _NB,
    ],
    compiler_params=pltpu.CompilerParams(use_tc_tiling_on_sc=False,
                                         needs_layout_passes=False),
  )


def _bn_from_stats(x, st, g, be):
    mu = st[0:1, :] * (1.0 / _N)
    var = st[1:2, :] * (1.0 / _N) - mu * mu
    inv = lax.rsqrt(var + 1e-5)
    return (x - mu) * inv * g + be


def _store_qkv(q, k, v, qao, qbo, kao, kbo, vao, vbo):
    qao[...] = q[:, :128]
    qbo[...] = q[:, 128:]
    kao[...] = k[:, :128]
    kbo[...] = k[:, 128:]
    vao[...] = v[:, :128]
    vbo[...] = v[:, 128:]


def _qkv0_body(x, qW, qb, kW, vW, qao, qbo, kao, kbo, vao, vbo):
    xb = x[...]
    q = (jnp.dot(xb, qW[...], preferred_element_type=jnp.float32)
         + qb[...]) * _SCALE
    k = jnp.dot(xb, kW[...], preferred_element_type=jnp.float32)
    v = jnp.dot(xb, vW[...], preferred_element_type=jnp.float32)
    _store_qkv(q, k, v, qao, qbo, kao, kbo, vao, vbo)


def _qkv1_body(z, st, g, be, qW, qb, kW, vW,
               xo, qao, qbo, kao, kbo, vao, vbo):
    xb = _bn_from_stats(z[...], st[...], g[...], be[...])
    xo[...] = xb
    q = (jnp.dot(xb, qW[...], preferred_element_type=jnp.float32)
         + qb[...]) * _SCALE
    k = jnp.dot(xb, kW[...], preferred_element_type=jnp.float32)
    v = jnp.dot(xb, vW[...], preferred_element_type=jnp.float32)
    _store_qkv(q, k, v, qao, qbo, kao, kbo, vao, vbo)


def _post_body(acca, accb, x, oW, ob, yo, sto):
    i = pl.program_id(0)
    pieces = []
    for a in (acca[...], accb[...]):
        for h in range(4):
            num = a[:, 32 * h:32 * h + 32]
            den = a[:, 128 + h][:, None] + 1e-16
            pieces.append(num / den)
    attn = jnp.concatenate(pieces, axis=1)
    y = jnp.dot(attn, oW[...], preferred_element_type=jnp.float32) + ob[...] + x[...]
    yo[...] = y

    @pl.when(i == 0)
    def _():
        sto[...] = jnp.zeros_like(sto)

    sto[...] += jnp.stack([jnp.sum(y, axis=0), jnp.sum(y * y, axis=0)])


def _ffn_body(y, st, w1, b1, w2, b2, g1, be1, zo, sto):
    i = pl.program_id(0)
    xb = _bn_from_stats(y[...], st[...], g1[...], be1[...])
    hh = jnp.maximum(jnp.dot(xb, w1[...], preferred_element_type=jnp.float32)
                     + b1[...], 0.0)
    z = jnp.dot(hh, w2[...], preferred_element_type=jnp.float32) + b2[...] + xb
    zo[...] = z

    @pl.when(i == 0)
    def _():
        sto[...] = jnp.zeros_like(sto)

    sto[...] += jnp.stack([jnp.sum(z, axis=0), jnp.sum(z * z, axis=0)])


def _final_body(z, st, g2, be2, lng, lnb, oo):
    xb = _bn_from_stats(z[...], st[...], g2[...], be2[...])
    mu = jnp.mean(xb, axis=1, keepdims=True)
    d = xb - mu
    var = jnp.mean(d * d, axis=1, keepdims=True)
    oo[...] = d * lax.rsqrt(var + 1e-5) * lng[...] + lnb[...]


def _full(shape):
    return pl.BlockSpec(shape, lambda i: tuple(0 for _ in shape))


def _rows(width):
    return pl.BlockSpec((_R, width), lambda i: (i, 0))


_f32 = jnp.float32


def _call_qkv0(x, qW, qb, kW, vW):
    return pl.pallas_call(
        _qkv0_body,
        grid=(_NR,),
        in_specs=[_rows(_D), _full((_D, _D)), _full((1, _D)),
                  _full((_D, _D)), _full((_D, _D))],
        out_specs=[_rows(128)] * 6,
        out_shape=[jax.ShapeDtypeStruct((_N, 128), _f32)] * 6,
    )(x, qW, qb.reshape(1, _D), kW, vW)


def _call_qkv1(z, st, g, be, qW, qb, kW, vW):
    return pl.pallas_call(
        _qkv1_body,
        grid=(_NR,),
        in_specs=[_rows(_D), _full((2, _D)), _full((1, _D)), _full((1, _D)),
                  _full((_D, _D)), _full((1, _D)), _full((_D, _D)),
                  _full((_D, _D))],
        out_specs=[_rows(_D)] + [_rows(128)] * 6,
        out_shape=([jax.ShapeDtypeStruct((_N, _D), _f32)]
                   + [jax.ShapeDtypeStruct((_N, 128), _f32)] * 6),
    )(z, st, g.reshape(1, _D), be.reshape(1, _D), qW, qb.reshape(1, _D), kW, vW)


def _call_post(acc, x, oW, ob):
    return pl.pallas_call(
        _post_body,
        grid=(_NR,),
        in_specs=[pl.BlockSpec((_R, _W), lambda i: (i, 0)),
                  pl.BlockSpec((_R, _W), lambda i: (i + _NR, 0)),
                  _rows(_D), _full((_D, _D)), _full((1, _D))],
        out_specs=[_rows(_D), _full((2, _D))],
        out_shape=[jax.ShapeDtypeStruct((_N, _D), _f32),
                   jax.ShapeDtypeStruct((2, _D), _f32)],
    )(acc, acc, x, oW, ob.reshape(1, _D))


def _call_ffn(y, st, w1, b1, w2, b2, g1, be1):
    return pl.pallas_call(
        _ffn_body,
        grid=(_NR,),
        in_specs=[_rows(_D), _full((2, _D)), _full((_D, _FF)), _full((1, _FF)),
                  _full((_FF, _D)), _full((1, _D)), _full((1, _D)),
                  _full((1, _D))],
        out_specs=[_rows(_D), _full((2, _D))],
        out_shape=[jax.ShapeDtypeStruct((_N, _D), _f32),
                   jax.ShapeDtypeStruct((2, _D), _f32)],
    )(y, st, w1, b1.reshape(1, _FF), w2, b2.reshape(1, _D),
      g1.reshape(1, _D), be1.reshape(1, _D))


def _call_final(z, st, g2, be2, lng, lnb):
    return pl.pallas_call(
        _final_body,
        grid=(_NR,),
        in_specs=[_rows(_D), _full((2, _D)), _full((1, _D)), _full((1, _D)),
                  _full((1, _D)), _full((1, _D))],
        out_specs=_rows(_D),
        out_shape=jax.ShapeDtypeStruct((_N, _D), _f32),
    )(z, st, g2.reshape(1, _D), be2.reshape(1, _D),
      lng.reshape(1, _D), lnb.reshape(1, _D))


def _edge(qkv, dst3, src3, zeros):
    return _edge_call()(*qkv, dst3, src3, zeros)


def kernel(src, edge_index,
           l0_qW, l0_qb, l0_kW, l0_vW, l0_oW, l0_ob, l0_w1, l0_b1, l0_w2,
           l0_b2, l0_g1, l0_be1, l0_g2, l0_be2,
           l1_qW, l1_qb, l1_kW, l1_vW, l1_oW, l1_ob, l1_w1, l1_b1, l1_w2,
           l1_b2, l1_g1, l1_be1, l1_g2, l1_be2,
           ln_g, ln_b):
    dst3 = edge_index[1].reshape(_NS, _NG, _G)
    src3 = edge_index[0].reshape(_NS, _NG, _G)
    zeros = jnp.zeros((_RPT, _W), jnp.float32)

    qkv = _call_qkv0(src, l0_qW, l0_qb, l0_kW, l0_vW)
    acc = _edge(qkv, dst3, src3, zeros)
    y, st = _call_post(acc, src, l0_oW, l0_ob)
    z, st = _call_ffn(y, st, l0_w1, l0_b1, l0_w2, l0_b2, l0_g1, l0_be1)

    xb, *qkv = _call_qkv1(z, st, l0_g2, l0_be2, l1_qW, l1_qb, l1_kW, l1_vW)
    acc = _edge(qkv, dst3, src3, zeros)
    y, st = _call_post(acc, xb, l1_oW, l1_ob)
    z, st = _call_ffn(y, st, l1_w1, l1_b1, l1_w2, l1_b2, l1_g1, l1_be1)

    return _call_final(z, st, l1_g2, l1_be2, ln_g, ln_b)


# speedup vs baseline: 1.0719x; 1.0100x over previous
import functools

import jax
import jax.numpy as jnp
from jax import lax
from jax.experimental import pallas as pl
from jax.experimental.pallas import tpu as pltpu
from jax.experimental.pallas import tpu_sc as plsc

_N = 10000
_E = 160000
_D = 256
_H = 8
_HD = 32
_FF = 1024
_SCALE = float(_HD) ** -0.5

_NC = 2
_NS = 16
_G = 16
_EC = _E // _NS
_NG = _EC // _G
_W = 136
_RPT = _N // _NS
_NB = 2

_R = 2000
_NR = _N // _R


def _edge_body(qa, qb, ka, kb, va, vb, dst4, src4, zeros, out,
               acc_sh, idx_dst, idx_src, qg, kg, vg, og,
               semq, semk, semv, sems):
    c = lax.axis_index("c")
    s = lax.axis_index("s")
    iota16 = lax.iota(jnp.int32, 16)
    zero16 = jnp.zeros((16,), jnp.float32)

    pltpu.sync_copy(dst4.at[s], idx_dst)
    pltpu.sync_copy(src4.at[s], idx_src)

    for b in range(_NB):
        def zrow(r, carry):
            for j in range(_W // 16):
                og[b, r, pl.ds(j * 16, 16)] = zero16
            og[b, r, pl.ds(_W - 16, 16)] = zero16
            return carry
        lax.fori_loop(0, _G, zrow, 0)

    base = s * _RPT
    pltpu.sync_copy(zeros, acc_sh.at[pl.ds(base, _RPT)])
    plsc.subcore_barrier()

    sem_by_slot = tuple(
        (semq[b], semk[b], semv[b], sems[b]) for b in range(_NB))

    def issue(g, b):
        sq, sk, sv_, _ = sem_by_slot[b]

        @pl.when(c == 0)
        def _():
            pltpu.async_copy(qa.at[idx_dst.at[g]], qg.at[b], sq)
            pltpu.async_copy(ka.at[idx_src.at[g]], kg.at[b], sk)
            pltpu.async_copy(va.at[idx_src.at[g]], vg.at[b], sv_)

        @pl.when(c == 1)
        def _():
            pltpu.async_copy(qb.at[idx_dst.at[g]], qg.at[b], sq)
            pltpu.async_copy(kb.at[idx_src.at[g]], kg.at[b], sk)
            pltpu.async_copy(vb.at[idx_src.at[g]], vg.at[b], sv_)

    def consume(g, b, last):
        sq, sk, sv_, ss = sem_by_slot[b]

        @pl.when(g >= _NB)
        def _():
            pltpu.make_async_copy(og.at[b], acc_sh.at[idx_dst.at[g]],
                                  ss).wait()

        pltpu.make_async_copy(qa.at[idx_dst.at[g]], qg.at[b], sq).wait()
        pltpu.make_async_copy(ka.at[idx_src.at[g]], kg.at[b], sk).wait()
        pltpu.make_async_copy(va.at[idx_src.at[g]], vg.at[b], sv_).wait()

        mask0 = iota16 == 0

        def edge(i, c1):
            es = (4 * i, 4 * i + 1, 4 * i + 2, 4 * i + 3)
            ps = []
            for e in es:
                for h in range(4):
                    p = (qg[b, e, pl.ds(h * 32, 16)] *
                         kg[b, e, pl.ds(h * 32, 16)]
                         + qg[b, e, pl.ds(h * 32 + 16, 16)] *
                         kg[b, e, pl.ds(h * 32 + 16, 16)])
                    ps.append(p)
            dots = [jnp.sum(p) for p in ps]
            for j, e in enumerate(es):
                for h in range(4):
                    we = jnp.exp(
                        lax.broadcast_in_dim(dots[4 * j + h], (16,), ()))
                    og[b, e, pl.ds(h * 32, 16)] = (
                        vg[b, e, pl.ds(h * 32, 16)] * we)
                    og[b, e, pl.ds(h * 32 + 16, 16)] = (
                        vg[b, e, pl.ds(h * 32 + 16, 16)] * we)
                    plsc.store_scatter(
                        og.at[b],
                        [jnp.full((16,), e, jnp.int32),
                         jnp.full((16,), 128 + h, jnp.int32)],
                        we, mask=mask0)
            return c1

        lax.fori_loop(0, _G // 4, edge, 0)

        pltpu.async_copy(og.at[b], acc_sh.at[idx_dst.at[g]], ss, add=True)

        if not last:
            @pl.when(g + _NB < _NG)
            def _():
                issue(g + _NB, b)

    for b in range(_NB):
        issue(b, b)

    def turn(j, carry):
        for b in range(_NB):
            consume(_NB * j + b, b, False)
        return carry

    lax.fori_loop(0, _NG // _NB, turn, 0)
    consume(_NG - 1, (_NG - 1) % _NB, True)

    for b in range(_NB):
        pltpu.make_async_copy(og.at[b], acc_sh.at[idx_dst.at[0]],
                              sems[b]).wait()
    plsc.subcore_barrier()

    pltpu.sync_copy(acc_sh.at[pl.ds(base, _RPT)],
                    out.at[pl.ds(c * _N + base, _RPT)])


@functools.cache
def _edge_call():
  return pl.kernel(
    _edge_body,
    out_type=jax.ShapeDtypeStruct((2 * _N, _W), jnp.float32),
    mesh=plsc.VectorSubcoreMesh(core_axis_name="c", subcore_axis_name="s",
                                num_cores=_NC, num_subcores=_NS),
    scratch_types=[
        pltpu.VMEM_SHARED((_N, _W), jnp.float32),
        pltpu.VMEM((_NG, _G), jnp.int32),
        pltpu.VMEM((_NG, _G), jnp.int32),
        pltpu.VMEM((_NB, _G, 128), jnp.float32),
        pltpu.VMEM((_NB, _G, 128), jnp.float32),
        pltpu.VMEM((_NB, _G, 128), jnp.float32),
        pltpu.VMEM((_NB, _G, _W), jnp.float32),
        [pltpu.SemaphoreType.DMA] * _NB,
        [pltpu.SemaphoreType.DMA] * _NB,
        [pltpu.SemaphoreType.DMA] * _NB,
        [pltpu.SemaphoreType.DMA] * _NB,
    ],
    compiler_params=pltpu.CompilerParams(use_tc_tiling_on_sc=False,
                                         needs_layout_passes=False),
  )


def _bn_from_stats(x, st, g, be):
    mu = st[0:1, :] * (1.0 / _N)
    var = st[1:2, :] * (1.0 / _N) - mu * mu
    inv = lax.rsqrt(var + 1e-5)
    return (x - mu) * inv * g + be


def _store_qkv(q, k, v, qao, qbo, kao, kbo, vao, vbo):
    qao[...] = q[:, :128]
    qbo[...] = q[:, 128:]
    kao[...] = k[:, :128]
    kbo[...] = k[:, 128:]
    vao[...] = v[:, :128]
    vbo[...] = v[:, 128:]


def _qkv0_body(x, qW, qb, kW, vW, qao, qbo, kao, kbo, vao, vbo):
    xb = x[...]
    q = (jnp.dot(xb, qW[...], preferred_element_type=jnp.float32)
         + qb[...]) * _SCALE
    k = jnp.dot(xb, kW[...], preferred_element_type=jnp.float32)
    v = jnp.dot(xb, vW[...], preferred_element_type=jnp.float32)
    _store_qkv(q, k, v, qao, qbo, kao, kbo, vao, vbo)


def _qkv1_body(z, st, g, be, qW, qb, kW, vW,
               xo, qao, qbo, kao, kbo, vao, vbo):
    xb = _bn_from_stats(z[...], st[...], g[...], be[...])
    xo[...] = xb
    q = (jnp.dot(xb, qW[...], preferred_element_type=jnp.float32)
         + qb[...]) * _SCALE
    k = jnp.dot(xb, kW[...], preferred_element_type=jnp.float32)
    v = jnp.dot(xb, vW[...], preferred_element_type=jnp.float32)
    _store_qkv(q, k, v, qao, qbo, kao, kbo, vao, vbo)


def _post_body(acca, accb, x, oW, ob, yo, sto):
    i = pl.program_id(0)
    pieces = []
    for a in (acca[...], accb[...]):
        for h in range(4):
            num = a[:, 32 * h:32 * h + 32]
            den = a[:, 128 + h][:, None] + 1e-16
            pieces.append(num / den)
    attn = jnp.concatenate(pieces, axis=1)
    y = jnp.dot(attn, oW[...], preferred_element_type=jnp.float32) + ob[...] + x[...]
    yo[...] = y

    @pl.when(i == 0)
    def _():
        sto[...] = jnp.zeros_like(sto)

    sto[...] += jnp.stack([jnp.sum(y, axis=0), jnp.sum(y * y, axis=0)])


def _ffn_body(y, st, w1, b1, w2, b2, g1, be1, zo, sto):
    i = pl.program_id(0)
    xb = _bn_from_stats(y[...], st[...], g1[...], be1[...])
    hh = jnp.maximum(jnp.dot(xb, w1[...], preferred_element_type=jnp.float32)
                     + b1[...], 0.0)
    z = jnp.dot(hh, w2[...], preferred_element_type=jnp.float32) + b2[...] + xb
    zo[...] = z

    @pl.when(i == 0)
    def _():
        sto[...] = jnp.zeros_like(sto)

    sto[...] += jnp.stack([jnp.sum(z, axis=0), jnp.sum(z * z, axis=0)])


def _final_body(z, st, g2, be2, lng, lnb, oo):
    xb = _bn_from_stats(z[...], st[...], g2[...], be2[...])
    mu = jnp.mean(xb, axis=1, keepdims=True)
    d = xb - mu
    var = jnp.mean(d * d, axis=1, keepdims=True)
    oo[...] = d * lax.rsqrt(var + 1e-5) * lng[...] + lnb[...]


def _full(shape):
    return pl.BlockSpec(shape, lambda i: tuple(0 for _ in shape))


def _rows(width):
    return pl.BlockSpec((_R, width), lambda i: (i, 0))


_f32 = jnp.float32


def _call_qkv0(x, qW, qb, kW, vW):
    return pl.pallas_call(
        _qkv0_body,
        grid=(_NR,),
        in_specs=[_rows(_D), _full((_D, _D)), _full((1, _D)),
                  _full((_D, _D)), _full((_D, _D))],
        out_specs=[_rows(128)] * 6,
        out_shape=[jax.ShapeDtypeStruct((_N, 128), _f32)] * 6,
    )(x, qW, qb.reshape(1, _D), kW, vW)


def _call_qkv1(z, st, g, be, qW, qb, kW, vW):
    return pl.pallas_call(
        _qkv1_body,
        grid=(_NR,),
        in_specs=[_rows(_D), _full((2, _D)), _full((1, _D)), _full((1, _D)),
                  _full((_D, _D)), _full((1, _D)), _full((_D, _D)),
                  _full((_D, _D))],
        out_specs=[_rows(_D)] + [_rows(128)] * 6,
        out_shape=([jax.ShapeDtypeStruct((_N, _D), _f32)]
                   + [jax.ShapeDtypeStruct((_N, 128), _f32)] * 6),
    )(z, st, g.reshape(1, _D), be.reshape(1, _D), qW, qb.reshape(1, _D), kW, vW)


def _call_post(acc, x, oW, ob):
    return pl.pallas_call(
        _post_body,
        grid=(_NR,),
        in_specs=[pl.BlockSpec((_R, _W), lambda i: (i, 0)),
                  pl.BlockSpec((_R, _W), lambda i: (i + _NR, 0)),
                  _rows(_D), _full((_D, _D)), _full((1, _D))],
        out_specs=[_rows(_D), _full((2, _D))],
        out_shape=[jax.ShapeDtypeStruct((_N, _D), _f32),
                   jax.ShapeDtypeStruct((2, _D), _f32)],
    )(acc, acc, x, oW, ob.reshape(1, _D))


def _call_ffn(y, st, w1, b1, w2, b2, g1, be1):
    return pl.pallas_call(
        _ffn_body,
        grid=(_NR,),
        in_specs=[_rows(_D), _full((2, _D)), _full((_D, _FF)), _full((1, _FF)),
                  _full((_FF, _D)), _full((1, _D)), _full((1, _D)),
                  _full((1, _D))],
        out_specs=[_rows(_D), _full((2, _D))],
        out_shape=[jax.ShapeDtypeStruct((_N, _D), _f32),
                   jax.ShapeDtypeStruct((2, _D), _f32)],
    )(y, st, w1, b1.reshape(1, _FF), w2, b2.reshape(1, _D),
      g1.reshape(1, _D), be1.reshape(1, _D))


def _call_final(z, st, g2, be2, lng, lnb):
    return pl.pallas_call(
        _final_body,
        grid=(_NR,),
        in_specs=[_rows(_D), _full((2, _D)), _full((1, _D)), _full((1, _D)),
                  _full((1, _D)), _full((1, _D))],
        out_specs=_rows(_D),
        out_shape=jax.ShapeDtypeStruct((_N, _D), _f32),
    )(z, st, g2.reshape(1, _D), be2.reshape(1, _D),
      lng.reshape(1, _D), lnb.reshape(1, _D))


def _edge(qkv, dst3, src3, zeros):
    return _edge_call()(*qkv, dst3, src3, zeros)


def kernel(src, edge_index,
           l0_qW, l0_qb, l0_kW, l0_vW, l0_oW, l0_ob, l0_w1, l0_b1, l0_w2,
           l0_b2, l0_g1, l0_be1, l0_g2, l0_be2,
           l1_qW, l1_qb, l1_kW, l1_vW, l1_oW, l1_ob, l1_w1, l1_b1, l1_w2,
           l1_b2, l1_g1, l1_be1, l1_g2, l1_be2,
           ln_g, ln_b):
    dst3 = edge_index[1].reshape(_NS, _NG, _G)
    src3 = edge_index[0].reshape(_NS, _NG, _G)
    zeros = jnp.zeros((_RPT, _W), jnp.float32)

    qkv = _call_qkv0(src, l0_qW, l0_qb, l0_kW, l0_vW)
    acc = _edge(qkv, dst3, src3, zeros)
    y, st = _call_post(acc, src, l0_oW, l0_ob)
    z, st = _call_ffn(y, st, l0_w1, l0_b1, l0_w2, l0_b2, l0_g1, l0_be1)

    xb, *qkv = _call_qkv1(z, st, l0_g2, l0_be2, l1_qW, l1_qb, l1_kW, l1_vW)
    acc = _edge(qkv, dst3, src3, zeros)
    y, st = _call_post(acc, xb, l1_oW, l1_ob)
    z, st = _call_ffn(y, st, l1_w1, l1_b1, l1_w2, l1_b2, l1_g1, l1_be1)

    return _call_final(z, st, l1_g2, l1_be2, ln_g, ln_b)
